# baseline (speedup 1.0000x reference)
import functools

import jax
import jax.numpy as jnp
from jax.experimental import pallas as pl
from jax.experimental.pallas import tpu as pltpu

_M_T = 512
_N_CHUNK = 1024


def _gate_kernel(x_ref, w1_ref, b1_ref, w2_ref, b2_ref,
                 weights_out, logits_out, ent_out, cv_out,
                 ent_acc, sw_acc, sw2_acc,
                 *, nm, tokens, gates):
    m = pl.program_id(0)

    xb = x_ref[...].astype(jnp.bfloat16)
    H = w1_ref.shape[1]
    nc = min(_N_CHUNK, H)
    logits = None
    for j in range(H // nc):
        sl = slice(j * nc, (j + 1) * nc)
        hj = jnp.dot(xb, w1_ref[:, sl], preferred_element_type=jnp.float32)
        hj = (hj + b1_ref[:, sl]).astype(jnp.bfloat16)
        gj = jax.nn.gelu(hj)
        lj = jnp.dot(gj, w2_ref[sl, :], preferred_element_type=jnp.float32)
        logits = lj if logits is None else logits + lj
    logits = logits + b2_ref[...]
    logits_out[...] = logits
    w = jax.nn.sigmoid(logits)
    weights_out[...] = w

    sw = jnp.sum(w, axis=0, keepdims=True)
    sw2 = jnp.sum(w * w, axis=0, keepdims=True)
    mx = jnp.max(logits, axis=-1, keepdims=True)
    e = jnp.exp(logits - mx)
    p = e / jnp.sum(e, axis=-1, keepdims=True)
    ent = -jnp.sum(p * jnp.log(p + 1e-9), axis=-1, keepdims=True)
    d = ent - jnp.log(jnp.float32(gates))
    e2 = jnp.sum(d * d).reshape(1, 1)

    @pl.when(m == 0)
    def _():
        sw_acc[...] = sw
        sw2_acc[...] = sw2
        ent_acc[...] = e2

    @pl.when(m != 0)
    def _():
        sw_acc[...] += sw
        sw2_acc[...] += sw2
        ent_acc[...] += e2

    @pl.when(m == nm - 1)
    def _():
        ent_out[...] = ent_acc[...] / tokens
        mean = sw_acc[...] / tokens
        var = sw2_acc[...] / tokens - mean * mean
        std = jnp.sqrt(jnp.maximum(var, 0.0))
        cv_out[...] = jnp.mean(std / (mean + 1e-9)).reshape(1, 1)


def kernel(hidden_states, W1, b1, W2, b2):
    B, S, H = hidden_states.shape
    G = W2.shape[1]
    M = B * S
    x = hidden_states.reshape(M, H)

    mt = min(_M_T, M)
    nm = M // mt

    w1b = W1.astype(jnp.bfloat16)
    w2b = W2.astype(jnp.bfloat16)
    b1r = b1.reshape(1, H)
    b2r = b2.reshape(1, G)

    out_shape = (
        jax.ShapeDtypeStruct((M, G), jnp.float32),
        jax.ShapeDtypeStruct((M, G), jnp.float32),
        jax.ShapeDtypeStruct((1, 1), jnp.float32),
        jax.ShapeDtypeStruct((1, 1), jnp.float32),
    )

    body = functools.partial(_gate_kernel, nm=nm, tokens=float(M), gates=G)

    weights, logits, ent, cv = pl.pallas_call(
        body,
        grid=(nm,),
        in_specs=[
            pl.BlockSpec((mt, H), lambda m: (m, 0)),
            pl.BlockSpec((H, H), lambda m: (0, 0)),
            pl.BlockSpec((1, H), lambda m: (0, 0)),
            pl.BlockSpec((H, G), lambda m: (0, 0)),
            pl.BlockSpec((1, G), lambda m: (0, 0)),
        ],
        out_specs=[
            pl.BlockSpec((mt, G), lambda m: (m, 0)),
            pl.BlockSpec((mt, G), lambda m: (m, 0)),
            pl.BlockSpec((1, 1), lambda m: (0, 0)),
            pl.BlockSpec((1, 1), lambda m: (0, 0)),
        ],
        out_shape=out_shape,
        scratch_shapes=[
            pltpu.VMEM((1, 1), jnp.float32),
            pltpu.VMEM((1, G), jnp.float32),
            pltpu.VMEM((1, G), jnp.float32),
        ],
        compiler_params=pltpu.CompilerParams(
            dimension_semantics=("arbitrary",),
        ),
    )(x, w1b, b1r, w2b, b2r)

    return (weights.reshape(B, S, G), logits.reshape(B, S, G),
            ent.reshape(()), cv.reshape(()))
